# Initial kernel scaffold; baseline (speedup 1.0000x reference)
import jax
import jax.numpy as jnp
from jax.experimental import pallas as pl


def kernel(shape_node_idx, shape_node_mult, edge_index, join_identities, num_nodes_hgraph, z_graph, shape_id_table, shape_mult_table, atom_id_table, Wq0, bq0, Wk0, bk0, Wv0, bv0, We0, Ws0, bs0, Wq1, bq1, Wk1, bk1, Wv1, bv1, We1, Ws1, bs1):
    raise NotImplementedError("write your pallas kernel here")


# speedup vs baseline: 4.5462x; 4.5462x over previous
import jax
import jax.numpy as jnp
from jax import lax
from jax.experimental import pallas as pl
from jax.experimental.pallas import tpu as pltpu
from jax.experimental.pallas import tpu_sc as plsc

N = 50000
C = 64
NC, NS = 2, 16
NW = NC * NS
CHUNK = 128
WV = 80
EPT = 25088
EPAD = EPT * NW
HALF = N // 2
ACC_ROWS = 25088
TRASH = 25080
NPT = 1568
NPAD = NPT * NW

_MESH = plsc.VectorSubcoreMesh(core_axis_name="c", subcore_axis_name="s")
_SC_PARAMS = pltpu.CompilerParams(use_tc_tiling_on_sc=False, needs_layout_passes=False)


def _embed_body(sidx, midx, stab, mtab, outa, outb, i_v, m_v, a_v, b_v, sem):
    wid = lax.axis_index("s") * NC + lax.axis_index("c")
    base = wid * NPT

    def chunk(ci, carry):
        cs = pl.multiple_of(base + ci * 112, 8)
        pltpu.sync_copy(sidx.at[pl.ds(cs, 112)], i_v)
        pltpu.sync_copy(midx.at[pl.ds(cs, 112)], m_v)
        pltpu.async_copy(stab.at[i_v], a_v, sem).wait()
        pltpu.async_copy(mtab.at[m_v], b_v, sem).wait()
        pltpu.sync_copy(a_v, outa.at[pl.ds(cs, 112)])
        pltpu.sync_copy(b_v, outb.at[pl.ds(cs, 112)])
        return carry

    lax.fori_loop(0, NPT // 112, chunk, 0)


def _embed(sidx, midx, stab, mtab):
    fn = pl.kernel(
        _embed_body,
        out_type=(
            jax.ShapeDtypeStruct((NPAD, 32), jnp.float32),
            jax.ShapeDtypeStruct((NPAD, 32), jnp.float32),
        ),
        mesh=_MESH,
        compiler_params=_SC_PARAMS,
        scratch_types=[
            pltpu.VMEM((112,), jnp.int32),
            pltpu.VMEM((112,), jnp.int32),
            pltpu.VMEM((112, 32), jnp.float32),
            pltpu.VMEM((112, 32), jnp.float32),
            pltpu.SemaphoreType.DMA,
        ],
    )
    return fn(sidx, midx, stab, mtab)


def _phase1_body(dstg, srcg, jidg, q, kv, etab, wv_out,
                 dst_v, src_v, jid_v, q_v, kv_v, e_v, out_v, sem):
    wid = lax.axis_index("s") * NC + lax.axis_index("c")
    base = wid * EPT
    onehot0 = (1 - jnp.minimum(lax.iota(jnp.int32, 16), 1)).astype(jnp.float32)

    def chunk(ci, carry):
        cs = pl.multiple_of(base + ci * CHUNK, 8)
        pltpu.sync_copy(dstg.at[pl.ds(cs, CHUNK)], dst_v)
        pltpu.sync_copy(srcg.at[pl.ds(cs, CHUNK)], src_v)
        pltpu.sync_copy(jidg.at[pl.ds(cs, CHUNK)], jid_v)
        pltpu.async_copy(q.at[dst_v], q_v, sem).wait()
        pltpu.async_copy(kv.at[src_v], kv_v, sem).wait()
        pltpu.async_copy(etab.at[jid_v], e_v, sem).wait()

        def edge(i, carry2):
            acc = jnp.zeros((16,), jnp.float32)
            for cc in range(4):
                qc = q_v[i, pl.ds(cc * 16, 16)]
                kc = kv_v[i, pl.ds(cc * 16, 16)]
                ec = e_v[i, pl.ds(cc * 16, 16)]
                acc = acc + qc * (kc + ec)
            s = jnp.sum(acc * 0.125)
            w16 = jnp.exp(jnp.full((16,), s, jnp.float32))
            for cc in range(4):
                vc = kv_v[i, pl.ds(64 + cc * 16, 16)]
                ec = e_v[i, pl.ds(cc * 16, 16)]
                out_v[i, pl.ds(cc * 16, 16)] = w16 * (vc + ec)
            out_v[i, pl.ds(64, 16)] = w16 * onehot0
            return carry2

        lax.fori_loop(0, CHUNK, edge, 0)
        pltpu.sync_copy(out_v, wv_out.at[pl.ds(cs, CHUNK)])
        return carry

    lax.fori_loop(0, EPT // CHUNK, chunk, 0)


def _phase1(dstg, srcg, jidg, q, kv, etab):
    fn = pl.kernel(
        _phase1_body,
        out_type=jax.ShapeDtypeStruct((EPAD, WV), jnp.float32),
        mesh=_MESH,
        compiler_params=_SC_PARAMS,
        scratch_types=[
            pltpu.VMEM((CHUNK,), jnp.int32),
            pltpu.VMEM((CHUNK,), jnp.int32),
            pltpu.VMEM((CHUNK,), jnp.int32),
            pltpu.VMEM((CHUNK, 64), jnp.float32),
            pltpu.VMEM((CHUNK, 128), jnp.float32),
            pltpu.VMEM((CHUNK, 64), jnp.float32),
            pltpu.VMEM((CHUNK, WV), jnp.float32),
            pltpu.SemaphoreType.DMA,
        ],
    )
    return fn(dstg, srcg, jidg, q, kv, etab)


QUARTER = N // 4
ACC2 = 12544
TRASH2 = 12504


def _phase2_body(p, dstp, wvin, nd_out, dst_v, lidx_v, wv_v, acc, sem):
    cid = lax.axis_index("c")
    sid = lax.axis_index("s")
    sc_base = (2 * p + cid) * QUARTER
    out_base = cid * QUARTER

    def zrow(i, carry):
        for cc in range(WV // 16):
            wv_v[i, pl.ds(cc * 16, 16)] = jnp.zeros((16,), jnp.float32)
        return carry

    lax.fori_loop(0, 112, zrow, 0)

    rows_per_tile = ACC2 // NS

    def zchunk(ci, carry):
        off = pl.multiple_of(sid * rows_per_tile + ci * 112, 8)
        pltpu.sync_copy(wv_v.at[pl.ds(0, 112)], acc.at[pl.ds(off, 112)])
        return carry

    lax.fori_loop(0, rows_per_tile // 112, zchunk, 0)
    plsc.subcore_barrier()

    ept2 = EPAD // NS
    base = sid * ept2

    def chunk(ci, carry):
        cs = pl.multiple_of(base + ci * CHUNK, 8)
        pltpu.sync_copy(dstp.at[pl.ds(cs, CHUNK)], dst_v)
        pltpu.sync_copy(wvin.at[pl.ds(cs, CHUNK)], wv_v)
        for g in range(CHUNK // 16):
            dv = dst_v[pl.ds(g * 16, 16)] - sc_base
            ok = (dv >= 0) & (dv < QUARTER)
            lidx_v[pl.ds(g * 16, 16)] = jnp.where(ok, dv, TRASH2)
        pltpu.sync_copy(wv_v, acc.at[lidx_v], add=True)
        return carry

    lax.fori_loop(0, ept2 // CHUNK, chunk, 0)
    plsc.subcore_barrier()

    nchunks = QUARTER // 125

    def dump(ci, carry):
        cidx = sid + NS * ci

        @pl.when(cidx < nchunks)
        def _():
            pltpu.sync_copy(acc.at[pl.ds(cidx * 125, 125)],
                            nd_out.at[pl.ds(out_base + cidx * 125, 125)])

        return carry

    lax.fori_loop(0, nchunks // NS + 1, dump, 0)


def _phase2(dstp, wv, p):
    fn = pl.kernel(
        lambda *refs: _phase2_body(p, *refs),
        out_type=jax.ShapeDtypeStruct((HALF, WV), jnp.float32),
        mesh=_MESH,
        compiler_params=_SC_PARAMS,
        scratch_types=[
            pltpu.VMEM((CHUNK,), jnp.int32),
            pltpu.VMEM((CHUNK,), jnp.int32),
            pltpu.VMEM((CHUNK, WV), jnp.float32),
            pltpu.VMEM_SHARED((ACC2, WV), jnp.float32),
            pltpu.SemaphoreType.DMA,
        ],
    )
    return fn(dstp, wv)


def _proj_kernel(x_ref, wq_ref, wkv_ref, ws_ref, bq_ref, bkv_ref, bs_ref,
                 q_ref, kv_ref, s_ref):
    x = x_ref[...]
    q_ref[...] = jnp.dot(x, wq_ref[...], preferred_element_type=jnp.float32) + bq_ref[0:1, :]
    kv_ref[...] = jnp.dot(x, wkv_ref[...], preferred_element_type=jnp.float32) + bkv_ref[0:1, :]
    s_ref[...] = jnp.dot(x, ws_ref[...], preferred_element_type=jnp.float32) + bs_ref[0:1, :]


def _proj(x, Wq, Wkv, Ws, bq, bkv, bs):
    n, fin = x.shape
    BLK = 400
    return pl.pallas_call(
        _proj_kernel,
        grid=(n // BLK,),
        in_specs=[
            pl.BlockSpec((BLK, fin), lambda i: (i, 0)),
            pl.BlockSpec((fin, 64), lambda i: (0, 0)),
            pl.BlockSpec((fin, 128), lambda i: (0, 0)),
            pl.BlockSpec((fin, 64), lambda i: (0, 0)),
            pl.BlockSpec((8, 64), lambda i: (0, 0)),
            pl.BlockSpec((8, 128), lambda i: (0, 0)),
            pl.BlockSpec((8, 64), lambda i: (0, 0)),
        ],
        out_specs=[
            pl.BlockSpec((BLK, 64), lambda i: (i, 0)),
            pl.BlockSpec((BLK, 128), lambda i: (i, 0)),
            pl.BlockSpec((BLK, 64), lambda i: (i, 0)),
        ],
        out_shape=[
            jax.ShapeDtypeStruct((n, 64), jnp.float32),
            jax.ShapeDtypeStruct((n, 128), jnp.float32),
            jax.ShapeDtypeStruct((n, 64), jnp.float32),
        ],
    )(x, Wq, Wkv, Ws, bq, bkv, bs)


def _etab_kernel(a_ref, w_ref, o_ref):
    o_ref[...] = jnp.dot(a_ref[...], w_ref[...], preferred_element_type=jnp.float32)


def _etab(atom_tab, We):
    return pl.pallas_call(
        _etab_kernel,
        out_shape=jax.ShapeDtypeStruct((atom_tab.shape[0], We.shape[1]), jnp.float32),
    )(atom_tab, We)


def _epi_kernel(nd_ref, s_ref, o_ref):
    num = nd_ref[:, 0:64]
    den = nd_ref[:, 64:65]
    o_ref[...] = jnp.maximum(num / (den + 1e-16) + s_ref[...], 0.0)


def _epi(nd, s):
    n = nd.shape[0]
    BLK = 400
    return pl.pallas_call(
        _epi_kernel,
        grid=(n // BLK,),
        in_specs=[
            pl.BlockSpec((BLK, WV), lambda i: (i, 0)),
            pl.BlockSpec((BLK, 64), lambda i: (i, 0)),
        ],
        out_specs=pl.BlockSpec((BLK, 64), lambda i: (i, 0)),
        out_shape=jax.ShapeDtypeStruct((n, 64), jnp.float32),
    )(nd, s)


def _pad_bias(b):
    return jnp.pad(b[None, :], ((0, 7), (0, 0)))


def kernel(shape_node_idx, shape_node_mult, edge_index, join_identities,
           num_nodes_hgraph, z_graph, shape_id_table, shape_mult_table,
           atom_id_table, Wq0, bq0, Wk0, bk0, Wv0, bv0, We0, Ws0, bs0,
           Wq1, bq1, Wk1, bk1, Wv1, bv1, We1, Ws1, bs1):
    src = edge_index[0].astype(jnp.int32)
    dst = edge_index[1].astype(jnp.int32)
    jid = (join_identities - 1).astype(jnp.int32)
    e = src.shape[0]
    pad = EPAD - e
    zpad = jnp.zeros((pad,), jnp.int32)
    dstp = jnp.concatenate([dst, jnp.full((pad,), N, jnp.int32)])
    dstg = jnp.concatenate([dst, zpad])
    srcp = jnp.concatenate([src, zpad])
    jidp = jnp.concatenate([jid, zpad])

    npad = NPAD - N
    sidxp = jnp.concatenate([shape_node_idx.astype(jnp.int32),
                             jnp.zeros((npad,), jnp.int32)])
    midxp = jnp.concatenate([shape_node_mult.astype(jnp.int32),
                             jnp.zeros((npad,), jnp.int32)])

    xa, xb = _embed(sidxp, midxp, shape_id_table, shape_mult_table)
    x0 = jnp.concatenate([xa[:N], xb[:N]], axis=1)

    layers = (
        (Wq0, bq0, Wk0, bk0, Wv0, bv0, We0, Ws0, bs0),
        (Wq1, bq1, Wk1, bk1, Wv1, bv1, We1, Ws1, bs1),
    )
    h = x0
    for (Wq, bq, Wk, bk, Wv, bv, We, Ws, bs) in layers:
        Wkv = jnp.concatenate([Wk, Wv], axis=1)
        bkv = jnp.concatenate([bk, bv])
        q, kv, sp = _proj(h, Wq, Wkv, Ws, _pad_bias(bq), _pad_bias(bkv),
                          _pad_bias(bs))
        etab = _etab(atom_id_table, We)
        wv = _phase1(dstg, srcp, jidp, q, kv, etab)
        nd = jnp.concatenate([_phase2(dstp, wv, 0), _phase2(dstp, wv, 1)])
        h = _epi(nd, sp)

    z_rep = jnp.repeat(z_graph, num_nodes_hgraph, axis=0,
                       total_repeat_length=N)
    return jnp.concatenate([x0, h, z_rep], axis=-1)


# speedup vs baseline: 4.5614x; 1.0033x over previous
import jax
import jax.numpy as jnp
from jax import lax
from jax.experimental import pallas as pl
from jax.experimental.pallas import tpu as pltpu
from jax.experimental.pallas import tpu_sc as plsc

N = 50000
C = 64
NC, NS = 2, 16
NW = NC * NS
CHUNK = 128
WV = 80
EPT = 25088
EPAD = EPT * NW
HALF = N // 2
ACC_ROWS = 25088
TRASH = 25080
NPT = 1568
NPAD = NPT * NW

_MESH = plsc.VectorSubcoreMesh(core_axis_name="c", subcore_axis_name="s")
_SC_PARAMS = pltpu.CompilerParams(use_tc_tiling_on_sc=False, needs_layout_passes=False)


def _embed_body(sidx, midx, stab, mtab, outa, outb, i_v, m_v, a_v, b_v, sem):
    wid = lax.axis_index("s") * NC + lax.axis_index("c")
    base = wid * NPT

    def chunk(ci, carry):
        cs = pl.multiple_of(base + ci * 112, 8)
        pltpu.sync_copy(sidx.at[pl.ds(cs, 112)], i_v)
        pltpu.sync_copy(midx.at[pl.ds(cs, 112)], m_v)
        pltpu.async_copy(stab.at[i_v], a_v, sem).wait()
        pltpu.async_copy(mtab.at[m_v], b_v, sem).wait()
        pltpu.sync_copy(a_v, outa.at[pl.ds(cs, 112)])
        pltpu.sync_copy(b_v, outb.at[pl.ds(cs, 112)])
        return carry

    lax.fori_loop(0, NPT // 112, chunk, 0)


def _embed(sidx, midx, stab, mtab):
    fn = pl.kernel(
        _embed_body,
        out_type=(
            jax.ShapeDtypeStruct((NPAD, 32), jnp.float32),
            jax.ShapeDtypeStruct((NPAD, 32), jnp.float32),
        ),
        mesh=_MESH,
        compiler_params=_SC_PARAMS,
        scratch_types=[
            pltpu.VMEM((112,), jnp.int32),
            pltpu.VMEM((112,), jnp.int32),
            pltpu.VMEM((112, 32), jnp.float32),
            pltpu.VMEM((112, 32), jnp.float32),
            pltpu.SemaphoreType.DMA,
        ],
    )
    return fn(sidx, midx, stab, mtab)


def _phase1_body(dstg, srcg, jidg, q, kv, etab, wv_out,
                 dst_v, src_v, jid_v, q_v, kv_v, e_v, out_v, sem):
    wid = lax.axis_index("s") * NC + lax.axis_index("c")
    base = wid * EPT
    onehot0 = (1 - jnp.minimum(lax.iota(jnp.int32, 16), 1)).astype(jnp.float32)

    def chunk(ci, carry):
        cs = pl.multiple_of(base + ci * CHUNK, 8)
        pltpu.sync_copy(dstg.at[pl.ds(cs, CHUNK)], dst_v)
        pltpu.sync_copy(srcg.at[pl.ds(cs, CHUNK)], src_v)
        pltpu.sync_copy(jidg.at[pl.ds(cs, CHUNK)], jid_v)
        pltpu.async_copy(q.at[dst_v], q_v, sem).wait()
        pltpu.async_copy(kv.at[src_v], kv_v, sem).wait()
        pltpu.async_copy(etab.at[jid_v], e_v, sem).wait()

        def edge(i4, carry2):
            for u in range(4):
                i = i4 * 4 + u
                acc = jnp.zeros((16,), jnp.float32)
                for cc in range(4):
                    qc = q_v[i, pl.ds(cc * 16, 16)]
                    kc = kv_v[i, pl.ds(cc * 16, 16)]
                    ec = e_v[i, pl.ds(cc * 16, 16)]
                    acc = acc + qc * (kc + ec)
                s = jnp.sum(acc * 0.125)
                w16 = jnp.exp(jnp.full((16,), s, jnp.float32))
                for cc in range(4):
                    vc = kv_v[i, pl.ds(64 + cc * 16, 16)]
                    ec = e_v[i, pl.ds(cc * 16, 16)]
                    out_v[i, pl.ds(cc * 16, 16)] = w16 * (vc + ec)
                out_v[i, pl.ds(64, 16)] = w16 * onehot0
            return carry2

        lax.fori_loop(0, CHUNK // 4, edge, 0)
        pltpu.sync_copy(out_v, wv_out.at[pl.ds(cs, CHUNK)])
        return carry

    lax.fori_loop(0, EPT // CHUNK, chunk, 0)


def _phase1(dstg, srcg, jidg, q, kv, etab):
    fn = pl.kernel(
        _phase1_body,
        out_type=jax.ShapeDtypeStruct((EPAD, WV), jnp.float32),
        mesh=_MESH,
        compiler_params=_SC_PARAMS,
        scratch_types=[
            pltpu.VMEM((CHUNK,), jnp.int32),
            pltpu.VMEM((CHUNK,), jnp.int32),
            pltpu.VMEM((CHUNK,), jnp.int32),
            pltpu.VMEM((CHUNK, 64), jnp.float32),
            pltpu.VMEM((CHUNK, 128), jnp.float32),
            pltpu.VMEM((CHUNK, 64), jnp.float32),
            pltpu.VMEM((CHUNK, WV), jnp.float32),
            pltpu.SemaphoreType.DMA,
        ],
    )
    return fn(dstg, srcg, jidg, q, kv, etab)


QUARTER = N // 4
ACC2 = 12544
TRASH2 = 12504


def _phase2_body(p, dstp, wvin, nd_out, dst_v, lidx_v, wv_v, acc, sem):
    cid = lax.axis_index("c")
    sid = lax.axis_index("s")
    sc_base = (2 * p + cid) * QUARTER
    out_base = cid * QUARTER

    def zrow(i, carry):
        for cc in range(WV // 16):
            wv_v[i, pl.ds(cc * 16, 16)] = jnp.zeros((16,), jnp.float32)
        return carry

    lax.fori_loop(0, 112, zrow, 0)

    rows_per_tile = ACC2 // NS

    def zchunk(ci, carry):
        off = pl.multiple_of(sid * rows_per_tile + ci * 112, 8)
        pltpu.sync_copy(wv_v.at[pl.ds(0, 112)], acc.at[pl.ds(off, 112)])
        return carry

    lax.fori_loop(0, rows_per_tile // 112, zchunk, 0)
    plsc.subcore_barrier()

    ept2 = EPAD // NS
    base = sid * ept2

    def chunk(ci, carry):
        cs = pl.multiple_of(base + ci * CHUNK, 8)
        pltpu.sync_copy(dstp.at[pl.ds(cs, CHUNK)], dst_v)
        pltpu.sync_copy(wvin.at[pl.ds(cs, CHUNK)], wv_v)
        for g in range(CHUNK // 16):
            dv = dst_v[pl.ds(g * 16, 16)] - sc_base
            ok = (dv >= 0) & (dv < QUARTER)
            lidx_v[pl.ds(g * 16, 16)] = jnp.where(ok, dv, TRASH2)
        pltpu.sync_copy(wv_v, acc.at[lidx_v], add=True)
        return carry

    lax.fori_loop(0, ept2 // CHUNK, chunk, 0)
    plsc.subcore_barrier()

    nchunks = QUARTER // 125

    def dump(ci, carry):
        cidx = sid + NS * ci

        @pl.when(cidx < nchunks)
        def _():
            pltpu.sync_copy(acc.at[pl.ds(cidx * 125, 125)],
                            nd_out.at[pl.ds(out_base + cidx * 125, 125)])

        return carry

    lax.fori_loop(0, nchunks // NS + 1, dump, 0)


def _phase2(dstp, wv, p):
    fn = pl.kernel(
        lambda *refs: _phase2_body(p, *refs),
        out_type=jax.ShapeDtypeStruct((HALF, WV), jnp.float32),
        mesh=_MESH,
        compiler_params=_SC_PARAMS,
        scratch_types=[
            pltpu.VMEM((CHUNK,), jnp.int32),
            pltpu.VMEM((CHUNK,), jnp.int32),
            pltpu.VMEM((CHUNK, WV), jnp.float32),
            pltpu.VMEM_SHARED((ACC2, WV), jnp.float32),
            pltpu.SemaphoreType.DMA,
        ],
    )
    return fn(dstp, wv)


def _proj_kernel(x_ref, wq_ref, wkv_ref, ws_ref, bq_ref, bkv_ref, bs_ref,
                 q_ref, kv_ref, s_ref):
    x = x_ref[...]
    q_ref[...] = jnp.dot(x, wq_ref[...], preferred_element_type=jnp.float32) + bq_ref[0:1, :]
    kv_ref[...] = jnp.dot(x, wkv_ref[...], preferred_element_type=jnp.float32) + bkv_ref[0:1, :]
    s_ref[...] = jnp.dot(x, ws_ref[...], preferred_element_type=jnp.float32) + bs_ref[0:1, :]


def _proj(x, Wq, Wkv, Ws, bq, bkv, bs):
    n, fin = x.shape
    BLK = 400
    return pl.pallas_call(
        _proj_kernel,
        grid=(n // BLK,),
        in_specs=[
            pl.BlockSpec((BLK, fin), lambda i: (i, 0)),
            pl.BlockSpec((fin, 64), lambda i: (0, 0)),
            pl.BlockSpec((fin, 128), lambda i: (0, 0)),
            pl.BlockSpec((fin, 64), lambda i: (0, 0)),
            pl.BlockSpec((8, 64), lambda i: (0, 0)),
            pl.BlockSpec((8, 128), lambda i: (0, 0)),
            pl.BlockSpec((8, 64), lambda i: (0, 0)),
        ],
        out_specs=[
            pl.BlockSpec((BLK, 64), lambda i: (i, 0)),
            pl.BlockSpec((BLK, 128), lambda i: (i, 0)),
            pl.BlockSpec((BLK, 64), lambda i: (i, 0)),
        ],
        out_shape=[
            jax.ShapeDtypeStruct((n, 64), jnp.float32),
            jax.ShapeDtypeStruct((n, 128), jnp.float32),
            jax.ShapeDtypeStruct((n, 64), jnp.float32),
        ],
    )(x, Wq, Wkv, Ws, bq, bkv, bs)


def _etab_kernel(a_ref, w_ref, o_ref):
    o_ref[...] = jnp.dot(a_ref[...], w_ref[...], preferred_element_type=jnp.float32)


def _etab(atom_tab, We):
    return pl.pallas_call(
        _etab_kernel,
        out_shape=jax.ShapeDtypeStruct((atom_tab.shape[0], We.shape[1]), jnp.float32),
    )(atom_tab, We)


def _epi_kernel(nd_ref, s_ref, o_ref):
    num = nd_ref[:, 0:64]
    den = nd_ref[:, 64:65]
    o_ref[...] = jnp.maximum(num / (den + 1e-16) + s_ref[...], 0.0)


def _epi(nd, s):
    n = nd.shape[0]
    BLK = 400
    return pl.pallas_call(
        _epi_kernel,
        grid=(n // BLK,),
        in_specs=[
            pl.BlockSpec((BLK, WV), lambda i: (i, 0)),
            pl.BlockSpec((BLK, 64), lambda i: (i, 0)),
        ],
        out_specs=pl.BlockSpec((BLK, 64), lambda i: (i, 0)),
        out_shape=jax.ShapeDtypeStruct((n, 64), jnp.float32),
    )(nd, s)


def _pad_bias(b):
    return jnp.pad(b[None, :], ((0, 7), (0, 0)))


def kernel(shape_node_idx, shape_node_mult, edge_index, join_identities,
           num_nodes_hgraph, z_graph, shape_id_table, shape_mult_table,
           atom_id_table, Wq0, bq0, Wk0, bk0, Wv0, bv0, We0, Ws0, bs0,
           Wq1, bq1, Wk1, bk1, Wv1, bv1, We1, Ws1, bs1):
    src = edge_index[0].astype(jnp.int32)
    dst = edge_index[1].astype(jnp.int32)
    jid = (join_identities - 1).astype(jnp.int32)
    e = src.shape[0]
    pad = EPAD - e
    zpad = jnp.zeros((pad,), jnp.int32)
    dstp = jnp.concatenate([dst, jnp.full((pad,), N, jnp.int32)])
    dstg = jnp.concatenate([dst, zpad])
    srcp = jnp.concatenate([src, zpad])
    jidp = jnp.concatenate([jid, zpad])

    npad = NPAD - N
    sidxp = jnp.concatenate([shape_node_idx.astype(jnp.int32),
                             jnp.zeros((npad,), jnp.int32)])
    midxp = jnp.concatenate([shape_node_mult.astype(jnp.int32),
                             jnp.zeros((npad,), jnp.int32)])

    xa, xb = _embed(sidxp, midxp, shape_id_table, shape_mult_table)
    x0 = jnp.concatenate([xa[:N], xb[:N]], axis=1)

    layers = (
        (Wq0, bq0, Wk0, bk0, Wv0, bv0, We0, Ws0, bs0),
        (Wq1, bq1, Wk1, bk1, Wv1, bv1, We1, Ws1, bs1),
    )
    h = x0
    for (Wq, bq, Wk, bk, Wv, bv, We, Ws, bs) in layers:
        Wkv = jnp.concatenate([Wk, Wv], axis=1)
        bkv = jnp.concatenate([bk, bv])
        q, kv, sp = _proj(h, Wq, Wkv, Ws, _pad_bias(bq), _pad_bias(bkv),
                          _pad_bias(bs))
        etab = _etab(atom_id_table, We)
        wv = _phase1(dstg, srcp, jidp, q, kv, etab)
        nd = jnp.concatenate([_phase2(dstp, wv, 0), _phase2(dstp, wv, 1)])
        h = _epi(nd, sp)

    z_rep = jnp.repeat(z_graph, num_nodes_hgraph, axis=0,
                       total_repeat_length=N)
    return jnp.concatenate([x0, h, z_rep], axis=-1)


# speedup vs baseline: 5.6008x; 1.2279x over previous
import jax
import jax.numpy as jnp
from jax import lax
from jax.experimental import pallas as pl
from jax.experimental.pallas import tpu as pltpu
from jax.experimental.pallas import tpu_sc as plsc

N = 50000
C = 64
NC, NS = 2, 16
NW = NC * NS
CHUNK = 128
WV = 80
EPT = 25088
EPAD = EPT * NW
HALF = N // 2
ACC_ROWS = 25088
TRASH = 25080
NPT = 1568
NPAD = NPT * NW

_MESH = plsc.VectorSubcoreMesh(core_axis_name="c", subcore_axis_name="s")
_SC_PARAMS = pltpu.CompilerParams(use_tc_tiling_on_sc=False, needs_layout_passes=False)


def _embed_body(sidx, midx, stab, mtab, outa, outb, i_v, m_v, a_v, b_v, sem):
    wid = lax.axis_index("s") * NC + lax.axis_index("c")
    base = wid * NPT

    def chunk(ci, carry):
        cs = pl.multiple_of(base + ci * 112, 8)
        pltpu.sync_copy(sidx.at[pl.ds(cs, 112)], i_v)
        pltpu.sync_copy(midx.at[pl.ds(cs, 112)], m_v)
        pltpu.async_copy(stab.at[i_v], a_v, sem).wait()
        pltpu.async_copy(mtab.at[m_v], b_v, sem).wait()
        pltpu.sync_copy(a_v, outa.at[pl.ds(cs, 112)])
        pltpu.sync_copy(b_v, outb.at[pl.ds(cs, 112)])
        return carry

    lax.fori_loop(0, NPT // 112, chunk, 0)


def _embed(sidx, midx, stab, mtab):
    fn = pl.kernel(
        _embed_body,
        out_type=(
            jax.ShapeDtypeStruct((NPAD, 32), jnp.float32),
            jax.ShapeDtypeStruct((NPAD, 32), jnp.float32),
        ),
        mesh=_MESH,
        compiler_params=_SC_PARAMS,
        scratch_types=[
            pltpu.VMEM((112,), jnp.int32),
            pltpu.VMEM((112,), jnp.int32),
            pltpu.VMEM((112, 32), jnp.float32),
            pltpu.VMEM((112, 32), jnp.float32),
            pltpu.SemaphoreType.DMA,
        ],
    )
    return fn(sidx, midx, stab, mtab)


def _phase1_body(dstg, srcg, jidg, q, kv, etab, wv_out,
                 dst0, src0, jid0, q0, kv0, e0, o0, sem0,
                 dst1, src1, jid1, q1, kv1, e1, o1, sem1):
    wid = lax.axis_index("s") * NC + lax.axis_index("c")
    base = wid * EPT
    onehot0 = (1 - jnp.minimum(lax.iota(jnp.int32, 16), 1)).astype(jnp.float32)
    nch = EPT // CHUNK

    def load_idx(c, dv, sv, jv):
        cs = pl.multiple_of(base + c * CHUNK, 8)
        pltpu.sync_copy(dstg.at[pl.ds(cs, CHUNK)], dv)
        pltpu.sync_copy(srcg.at[pl.ds(cs, CHUNK)], sv)
        pltpu.sync_copy(jidg.at[pl.ds(cs, CHUNK)], jv)

    def start(dv, sv, jv, qv, kvv, ev, sem):
        pltpu.async_copy(q.at[dv], qv, sem)
        pltpu.async_copy(kv.at[sv], kvv, sem)
        pltpu.async_copy(etab.at[jv], ev, sem)

    def drain(dv, sv, jv, qv, kvv, ev, sem):
        pltpu.make_async_copy(q.at[dv], qv, sem).wait()
        pltpu.make_async_copy(kv.at[sv], kvv, sem).wait()
        pltpu.make_async_copy(etab.at[jv], ev, sem).wait()

    def compute(c, qv, kvv, ev, ov):
        def edge(i4, carry2):
            for u in range(4):
                i = i4 * 4 + u
                acc = jnp.zeros((16,), jnp.float32)
                for cc in range(4):
                    qc = qv[i, pl.ds(cc * 16, 16)]
                    kc = kvv[i, pl.ds(cc * 16, 16)]
                    ec = ev[i, pl.ds(cc * 16, 16)]
                    acc = acc + qc * (kc + ec)
                s = jnp.sum(acc * 0.125)
                w16 = jnp.exp(jnp.full((16,), s, jnp.float32))
                for cc in range(4):
                    vc = kvv[i, pl.ds(64 + cc * 16, 16)]
                    ec = ev[i, pl.ds(cc * 16, 16)]
                    ov[i, pl.ds(cc * 16, 16)] = w16 * (vc + ec)
                ov[i, pl.ds(64, 16)] = w16 * onehot0
            return carry2

        lax.fori_loop(0, CHUNK // 4, edge, 0)
        cs = pl.multiple_of(base + c * CHUNK, 8)
        pltpu.sync_copy(ov, wv_out.at[pl.ds(cs, CHUNK)])

    load_idx(0, dst0, src0, jid0)
    start(dst0, src0, jid0, q0, kv0, e0, sem0)

    def pair(i, carry):
        ca = 2 * i
        cb = 2 * i + 1
        load_idx(cb, dst1, src1, jid1)
        start(dst1, src1, jid1, q1, kv1, e1, sem1)
        drain(dst0, src0, jid0, q0, kv0, e0, sem0)
        compute(ca, q0, kv0, e0, o0)

        @pl.when(cb + 1 < nch)
        def _():
            load_idx(cb + 1, dst0, src0, jid0)
            start(dst0, src0, jid0, q0, kv0, e0, sem0)

        drain(dst1, src1, jid1, q1, kv1, e1, sem1)
        compute(cb, q1, kv1, e1, o1)
        return carry

    lax.fori_loop(0, nch // 2, pair, 0)


def _phase1(dstg, srcg, jidg, q, kv, etab):
    bufs = [
        pltpu.VMEM((CHUNK,), jnp.int32),
        pltpu.VMEM((CHUNK,), jnp.int32),
        pltpu.VMEM((CHUNK,), jnp.int32),
        pltpu.VMEM((CHUNK, 64), jnp.float32),
        pltpu.VMEM((CHUNK, 128), jnp.float32),
        pltpu.VMEM((CHUNK, 64), jnp.float32),
        pltpu.VMEM((CHUNK, WV), jnp.float32),
        pltpu.SemaphoreType.DMA,
    ]
    fn = pl.kernel(
        _phase1_body,
        out_type=jax.ShapeDtypeStruct((EPAD, WV), jnp.float32),
        mesh=_MESH,
        compiler_params=_SC_PARAMS,
        scratch_types=bufs + bufs,
    )
    return fn(dstg, srcg, jidg, q, kv, etab)


QUARTER = N // 4
ACC2 = 12544
TRASH2 = 12504


def _phase2_body(p, dstp, wvin, nd_out, dst_v, lidx_v, wv_v, acc, sem):
    cid = lax.axis_index("c")
    sid = lax.axis_index("s")
    sc_base = (2 * p + cid) * QUARTER
    out_base = cid * QUARTER

    def zrow(i, carry):
        for cc in range(WV // 16):
            wv_v[i, pl.ds(cc * 16, 16)] = jnp.zeros((16,), jnp.float32)
        return carry

    lax.fori_loop(0, 112, zrow, 0)

    rows_per_tile = ACC2 // NS

    def zchunk(ci, carry):
        off = pl.multiple_of(sid * rows_per_tile + ci * 112, 8)
        pltpu.sync_copy(wv_v.at[pl.ds(0, 112)], acc.at[pl.ds(off, 112)])
        return carry

    lax.fori_loop(0, rows_per_tile // 112, zchunk, 0)
    plsc.subcore_barrier()

    ept2 = EPAD // NS
    base = sid * ept2

    def chunk(ci, carry):
        cs = pl.multiple_of(base + ci * CHUNK, 8)
        pltpu.sync_copy(dstp.at[pl.ds(cs, CHUNK)], dst_v)
        pltpu.sync_copy(wvin.at[pl.ds(cs, CHUNK)], wv_v)
        for g in range(CHUNK // 16):
            dv = dst_v[pl.ds(g * 16, 16)] - sc_base
            ok = (dv >= 0) & (dv < QUARTER)
            lidx_v[pl.ds(g * 16, 16)] = jnp.where(ok, dv, TRASH2)
        pltpu.sync_copy(wv_v, acc.at[lidx_v], add=True)
        return carry

    lax.fori_loop(0, ept2 // CHUNK, chunk, 0)
    plsc.subcore_barrier()

    nchunks = QUARTER // 125

    def dump(ci, carry):
        cidx = sid + NS * ci

        @pl.when(cidx < nchunks)
        def _():
            pltpu.sync_copy(acc.at[pl.ds(cidx * 125, 125)],
                            nd_out.at[pl.ds(out_base + cidx * 125, 125)])

        return carry

    lax.fori_loop(0, nchunks // NS + 1, dump, 0)


def _phase2(dstp, wv, p):
    fn = pl.kernel(
        lambda *refs: _phase2_body(p, *refs),
        out_type=jax.ShapeDtypeStruct((HALF, WV), jnp.float32),
        mesh=_MESH,
        compiler_params=_SC_PARAMS,
        scratch_types=[
            pltpu.VMEM((CHUNK,), jnp.int32),
            pltpu.VMEM((CHUNK,), jnp.int32),
            pltpu.VMEM((CHUNK, WV), jnp.float32),
            pltpu.VMEM_SHARED((ACC2, WV), jnp.float32),
            pltpu.SemaphoreType.DMA,
        ],
    )
    return fn(dstp, wv)


def _proj_kernel(x_ref, wq_ref, wkv_ref, ws_ref, bq_ref, bkv_ref, bs_ref,
                 q_ref, kv_ref, s_ref):
    x = x_ref[...]
    q_ref[...] = jnp.dot(x, wq_ref[...], preferred_element_type=jnp.float32) + bq_ref[0:1, :]
    kv_ref[...] = jnp.dot(x, wkv_ref[...], preferred_element_type=jnp.float32) + bkv_ref[0:1, :]
    s_ref[...] = jnp.dot(x, ws_ref[...], preferred_element_type=jnp.float32) + bs_ref[0:1, :]


def _proj(x, Wq, Wkv, Ws, bq, bkv, bs):
    n, fin = x.shape
    BLK = 400
    return pl.pallas_call(
        _proj_kernel,
        grid=(n // BLK,),
        in_specs=[
            pl.BlockSpec((BLK, fin), lambda i: (i, 0)),
            pl.BlockSpec((fin, 64), lambda i: (0, 0)),
            pl.BlockSpec((fin, 128), lambda i: (0, 0)),
            pl.BlockSpec((fin, 64), lambda i: (0, 0)),
            pl.BlockSpec((8, 64), lambda i: (0, 0)),
            pl.BlockSpec((8, 128), lambda i: (0, 0)),
            pl.BlockSpec((8, 64), lambda i: (0, 0)),
        ],
        out_specs=[
            pl.BlockSpec((BLK, 64), lambda i: (i, 0)),
            pl.BlockSpec((BLK, 128), lambda i: (i, 0)),
            pl.BlockSpec((BLK, 64), lambda i: (i, 0)),
        ],
        out_shape=[
            jax.ShapeDtypeStruct((n, 64), jnp.float32),
            jax.ShapeDtypeStruct((n, 128), jnp.float32),
            jax.ShapeDtypeStruct((n, 64), jnp.float32),
        ],
    )(x, Wq, Wkv, Ws, bq, bkv, bs)


def _etab_kernel(a_ref, w_ref, o_ref):
    o_ref[...] = jnp.dot(a_ref[...], w_ref[...], preferred_element_type=jnp.float32)


def _etab(atom_tab, We):
    return pl.pallas_call(
        _etab_kernel,
        out_shape=jax.ShapeDtypeStruct((atom_tab.shape[0], We.shape[1]), jnp.float32),
    )(atom_tab, We)


def _epi_kernel(nd_ref, s_ref, o_ref):
    num = nd_ref[:, 0:64]
    den = nd_ref[:, 64:65]
    o_ref[...] = jnp.maximum(num / (den + 1e-16) + s_ref[...], 0.0)


def _epi(nd, s):
    n = nd.shape[0]
    BLK = 400
    return pl.pallas_call(
        _epi_kernel,
        grid=(n // BLK,),
        in_specs=[
            pl.BlockSpec((BLK, WV), lambda i: (i, 0)),
            pl.BlockSpec((BLK, 64), lambda i: (i, 0)),
        ],
        out_specs=pl.BlockSpec((BLK, 64), lambda i: (i, 0)),
        out_shape=jax.ShapeDtypeStruct((n, 64), jnp.float32),
    )(nd, s)


def _pad_bias(b):
    return jnp.pad(b[None, :], ((0, 7), (0, 0)))


def kernel(shape_node_idx, shape_node_mult, edge_index, join_identities,
           num_nodes_hgraph, z_graph, shape_id_table, shape_mult_table,
           atom_id_table, Wq0, bq0, Wk0, bk0, Wv0, bv0, We0, Ws0, bs0,
           Wq1, bq1, Wk1, bk1, Wv1, bv1, We1, Ws1, bs1):
    src = edge_index[0].astype(jnp.int32)
    dst = edge_index[1].astype(jnp.int32)
    jid = (join_identities - 1).astype(jnp.int32)
    e = src.shape[0]
    pad = EPAD - e
    zpad = jnp.zeros((pad,), jnp.int32)
    dstp = jnp.concatenate([dst, jnp.full((pad,), N, jnp.int32)])
    dstg = jnp.concatenate([dst, zpad])
    srcp = jnp.concatenate([src, zpad])
    jidp = jnp.concatenate([jid, zpad])

    npad = NPAD - N
    sidxp = jnp.concatenate([shape_node_idx.astype(jnp.int32),
                             jnp.zeros((npad,), jnp.int32)])
    midxp = jnp.concatenate([shape_node_mult.astype(jnp.int32),
                             jnp.zeros((npad,), jnp.int32)])

    xa, xb = _embed(sidxp, midxp, shape_id_table, shape_mult_table)
    x0 = jnp.concatenate([xa[:N], xb[:N]], axis=1)

    layers = (
        (Wq0, bq0, Wk0, bk0, Wv0, bv0, We0, Ws0, bs0),
        (Wq1, bq1, Wk1, bk1, Wv1, bv1, We1, Ws1, bs1),
    )
    h = x0
    for (Wq, bq, Wk, bk, Wv, bv, We, Ws, bs) in layers:
        Wkv = jnp.concatenate([Wk, Wv], axis=1)
        bkv = jnp.concatenate([bk, bv])
        q, kv, sp = _proj(h, Wq, Wkv, Ws, _pad_bias(bq), _pad_bias(bkv),
                          _pad_bias(bs))
        etab = _etab(atom_id_table, We)
        wv = _phase1(dstg, srcp, jidp, q, kv, etab)
        nd = jnp.concatenate([_phase2(dstp, wv, 0), _phase2(dstp, wv, 1)])
        h = _epi(nd, sp)

    z_rep = jnp.repeat(z_graph, num_nodes_hgraph, axis=0,
                       total_repeat_length=N)
    return jnp.concatenate([x0, h, z_rep], axis=-1)
